# baseline (speedup 1.0000x reference)
import functools

import jax
import jax.numpy as jnp
from jax import lax
from jax.experimental import pallas as pl
from jax.experimental.pallas import tpu as pltpu
from jax.experimental.pallas import tpu_sc as plsc

NC = 2
NS = 16
NW = NC * NS
L = 16

B = 16384
D = 32
BPW = B // NW
CHUNK = 128
NCHUNK = BPW // CHUNK
GROUPS = BPW // L


def _mf_body(uid_hbm, iid_hbm, uet_hbm, ubt_hbm, iet_hbm, ibt_hbm,
             out_hbm, uid_v, iid_v, urt, irt, ub, ib, out_v, sem, bsem):
    wid = lax.axis_index("s") * NC + lax.axis_index("c")
    base = wid * BPW

    pltpu.sync_copy(uid_hbm.at[pl.ds(base, BPW)], uid_v)
    pltpu.sync_copy(iid_hbm.at[pl.ds(base, BPW)], iid_v)

    bias_copies = []
    for c in range(NCHUNK):
        sl = pl.ds(c * CHUNK, CHUNK)
        bias_copies.append(
            pltpu.async_copy(ubt_hbm.at[0].at[uid_v.at[sl]], ub.at[sl], bsem))
        bias_copies.append(
            pltpu.async_copy(ibt_hbm.at[0].at[iid_v.at[sl]], ib.at[sl], bsem))

    for d in range(D):
        up = uet_hbm.at[d]
        ip = iet_hbm.at[d]
        for c in range(NCHUNK):
            sl = pl.ds(c * CHUNK, CHUNK)
            pltpu.async_copy(up.at[uid_v.at[sl]], urt.at[d, sl], sem)
            pltpu.async_copy(ip.at[iid_v.at[sl]], irt.at[d, sl], sem)

    for cp in bias_copies:
        cp.wait()
    pltpu.make_async_copy(uet_hbm.at[:, pl.ds(0, BPW)], urt, sem).wait()
    pltpu.make_async_copy(iet_hbm.at[:, pl.ds(0, BPW)], irt, sem).wait()

    def group(g, carry):
        r0 = pl.multiple_of(g * L, L)
        sl = pl.ds(r0, L)
        acc = ub[sl] + ib[sl]
        for d in range(D):
            acc = acc + urt[d, sl] * irt[d, sl]
        out_v[sl] = acc
        return carry

    lax.fori_loop(0, GROUPS, group, 0)
    pltpu.sync_copy(out_v, out_hbm.at[pl.ds(base, BPW)])


_mf_kernel = functools.partial(
    pl.kernel,
    out_type=jax.ShapeDtypeStruct((B,), jnp.float32),
    mesh=plsc.VectorSubcoreMesh(
        core_axis_name="c", subcore_axis_name="s",
        num_cores=NC, num_subcores=NS),
    scratch_types=[
        pltpu.VMEM((BPW,), jnp.int32),
        pltpu.VMEM((BPW,), jnp.int32),
        pltpu.VMEM((D, BPW), jnp.float32),
        pltpu.VMEM((D, BPW), jnp.float32),
        pltpu.VMEM((BPW,), jnp.float32),
        pltpu.VMEM((BPW,), jnp.float32),
        pltpu.VMEM((BPW,), jnp.float32),
        pltpu.SemaphoreType.DMA,
        pltpu.SemaphoreType.DMA,
    ],
    compiler_params=pltpu.CompilerParams(needs_layout_passes=False,
                                         use_tc_tiling_on_sc=False),
)(_mf_body)


@jax.jit
def kernel(user_id, item_id, user_embedding, user_bias, item_embedding,
           item_bias):
    uid = user_id.astype(jnp.int32)
    iid = item_id.astype(jnp.int32)
    return _mf_kernel(uid, iid, user_embedding.T, user_bias.T,
                      item_embedding.T, item_bias.T)


# speedup vs baseline: 7.8449x; 7.8449x over previous
import functools

import jax
import jax.numpy as jnp
from jax import lax
from jax.experimental import pallas as pl
from jax.experimental.pallas import tpu as pltpu
from jax.experimental.pallas import tpu_sc as plsc
from jax.experimental import layout as jlayout

NC = 2
NS = 16
NW = NC * NS
L = 16

B = 16384
D = 32
BPW = B // NW
CHUNK = 128
NCHUNK = BPW // CHUNK
GROUPS = BPW // L


def _mf_body(uid_hbm, iid_hbm, uemb_hbm, ubf_hbm, iemb_hbm, ibf_hbm,
             out_hbm, uid_v, iid_v, urows, irows, ub, ib, mt, out_v, sem):
    wid = lax.axis_index("s") * NC + lax.axis_index("c")
    base = wid * BPW

    for c in range(NCHUNK):
        pltpu.sync_copy(uid_hbm.at[pl.ds(base + c * CHUNK, CHUNK)], uid_v.at[c])
        pltpu.sync_copy(iid_hbm.at[pl.ds(base + c * CHUNK, CHUNK)], iid_v.at[c])

    copies = []
    for c in range(NCHUNK):
        sl = pl.ds(c * CHUNK, CHUNK)
        copies.append(pltpu.async_copy(uemb_hbm.at[uid_v.at[c]], urows.at[sl], sem))
        copies.append(pltpu.async_copy(iemb_hbm.at[iid_v.at[c]], irows.at[sl], sem))
        copies.append(pltpu.async_copy(ubf_hbm.at[uid_v.at[c]], ub.at[sl], sem))
        copies.append(pltpu.async_copy(ibf_hbm.at[iid_v.at[c]], ib.at[sl], sem))
    for cp in copies:
        cp.wait()

    lanes = lax.iota(jnp.int32, L)

    def group(g, carry):
        r0 = pl.multiple_of(g * L, L)
        for r in range(L):
            row = r0 + r
            p0 = urows[row, pl.ds(0, L)]
            p1 = urows[row, pl.ds(L, L)]
            q0 = irows[row, pl.ds(0, L)]
            q1 = irows[row, pl.ds(L, L)]
            a = p0 * q0 + p1 * q1
            plsc.store_scatter(mt, [lanes, jnp.full((L,), r, jnp.int32)], a)
        acc = ub[pl.ds(r0, L)] + ib[pl.ds(r0, L)]
        for j in range(L):
            acc = acc + mt[j, pl.ds(0, L)]
        out_v[pl.ds(r0, L)] = acc
        return carry

    lax.fori_loop(0, GROUPS, group, 0)
    pltpu.sync_copy(out_v, out_hbm.at[pl.ds(base, BPW)])


_mf_kernel = functools.partial(
    pl.kernel,
    out_type=jax.ShapeDtypeStruct((B,), jnp.float32),
    mesh=plsc.VectorSubcoreMesh(
        core_axis_name="c", subcore_axis_name="s",
        num_cores=NC, num_subcores=NS),
    scratch_types=[
        pltpu.VMEM((NCHUNK, CHUNK), jnp.int32),
        pltpu.VMEM((NCHUNK, CHUNK), jnp.int32),
        pltpu.VMEM((BPW, D), jnp.float32),
        pltpu.VMEM((BPW, D), jnp.float32),
        pltpu.VMEM((BPW,), jnp.float32),
        pltpu.VMEM((BPW,), jnp.float32),
        pltpu.VMEM((L, L), jnp.float32),
        pltpu.VMEM((BPW,), jnp.float32),
        pltpu.SemaphoreType.DMA,
    ],
    compiler_params=pltpu.CompilerParams(needs_layout_passes=False,
                                         use_tc_tiling_on_sc=False),
)(_mf_body)

def _row_major(table, zbit):
    bits = lax.bitcast_convert_type(table, jnp.int32) ^ zbit
    x = lax.bitcast_convert_type(bits, jnp.float32)
    return jlayout.with_layout_constraint(
        x, jlayout.Layout((0, 1), ((8, 128),)))


@jax.jit
def kernel(user_id, item_id, user_embedding, user_bias, item_embedding,
           item_bias):
    uid = user_id.astype(jnp.int32)
    iid = item_id.astype(jnp.int32)
    z = uid[0] & 0
    return _mf_kernel(uid, iid, _row_major(user_embedding, z),
                      user_bias.reshape(-1),
                      _row_major(item_embedding, z),
                      item_bias.reshape(-1))
